# baseline (speedup 1.0000x reference)
import functools

import jax
import jax.numpy as jnp
from jax import lax
from jax.experimental import pallas as pl
from jax.experimental.pallas import tpu as pltpu
from jax.experimental.pallas import tpu_sc as plsc

N_NODES_C = 10000
N_EDGES_C = 320000
D_C = 128
DH = D_C // 2

NC = 2
NS = 16
EPT = N_EDGES_C // NS
KB = 80
NBB = EPT // KB
N_PAD = 10240
RPT = N_PAD // NS
GI = 10
GO = 2
SBB = 10
SBN = NBB // SBB
SBE = SBB * KB
NIB = 3


def _lane_splat(vec, lane):
  return lax.gather(
      vec, jnp.full((16, 1), lane, jnp.int32),
      dimension_numbers=lax.GatherDimensionNumbers(
          offset_dims=(), collapsed_slice_dims=(0,), start_index_map=(0,)),
      slice_sizes=(1,),
      mode=lax.GatherScatterMode.PROMISE_IN_BOUNDS)


def _sc_spmm(xflat, src_r, dst_r, adj_r, zeros_rows):
  mesh = plsc.VectorSubcoreMesh(core_axis_name="c", subcore_axis_name="s")

  @functools.partial(
      pl.kernel,
      mesh=mesh,
      compiler_params=pltpu.CompilerParams(use_tc_tiling_on_sc=False),
      out_type=jax.ShapeDtypeStruct((NC, N_PAD, DH), jnp.float32),
      scratch_types=[
          pltpu.VMEM((NIB, SBE), jnp.int32),
          pltpu.VMEM((NIB, SBB, KB), jnp.int32),
          pltpu.VMEM((NIB, SBB, KB), jnp.float32),
          pltpu.VMEM((GI, KB, DH // 2), jnp.int32),
          pltpu.VMEM((GO, KB, DH), jnp.float32),
          pltpu.VMEM_SHARED((N_PAD, DH), jnp.float32),
      ] + [pltpu.SemaphoreType.DMA] * (GI + GO + 2),
  )
  def spmm_kernel(x_hbm, src_hbm, dst_hbm, adj_hbm, z_hbm, out_hbm,
                  src_v, dst_v, adj_v, rin_v, rout_v, acc_sh, *sems):
    c = lax.axis_index("c")
    s = lax.axis_index("s")
    gsem = sems[:GI]
    ssem = sems[GI:GI + GO]
    ksem = sems[GI + GO]
    zsem = sems[GI + GO + 1]

    offv = jnp.full((16,), c, jnp.int32)

    def stage_idx(sb, par):
      pltpu.make_async_copy(src_hbm.at[s, sb], src_v.at[par], ksem).start()
      pltpu.make_async_copy(dst_hbm.at[s, sb], dst_v.at[par], ksem).start()
      pltpu.make_async_copy(adj_hbm.at[s, sb], adj_v.at[par], ksem).start()

    def wait_idx(sb, par):
      pltpu.make_async_copy(src_hbm.at[s, sb], src_v.at[par], ksem).wait()
      pltpu.make_async_copy(dst_hbm.at[s, sb], dst_v.at[par], ksem).wait()
      pltpu.make_async_copy(adj_hbm.at[s, sb], adj_v.at[par], ksem).wait()

    def offset_src(par):
      def off_body(i, carry):
        sl = pl.ds(i * 16, 16)
        src_v[par, sl] = src_v[par, sl] * 2 + offv
        return carry
      lax.fori_loop(0, SBE // 16, off_body, 0)

    def gather_cp(par, bb, i):
      return pltpu.make_async_copy(
          x_hbm.at[src_v.at[par, pl.ds(bb * KB, KB)]], rin_v.at[i], gsem[i])

    def scatter_cp(par, bb, q):
      return pltpu.make_async_copy(rout_v.at[q], acc_sh.at[dst_v.at[par, bb]],
                                   ssem[q])

    zcp = pltpu.async_copy(z_hbm, acc_sh.at[pl.ds(s * RPT, RPT)], zsem)
    stage_idx(0, 0)
    stage_idx(1, 1)
    wait_idx(0, 0)
    wait_idx(1, 1)
    offset_src(0)
    offset_src(1)
    for i in range(GI):
      gather_cp(0, i, i).start()
    zcp.wait()
    plsc.subcore_barrier()

    def sb_body(sb, carry):
      par = lax.rem(sb, NIB)
      nx1 = lax.rem(sb + 1, NIB)
      nx2 = lax.rem(sb + 2, NIB)
      not_last = sb < SBN - 1
      for bb in range(SBB):
        gi = bb % GI
        q = bb % GO
        gather_cp(par, bb, gi).wait()
        if bb >= GO:
          scatter_cp(par, bb - GO, q).wait()
        else:
          @pl.when(sb > 0)
          def _(nx2=nx2, bb=bb, q=q):
            scatter_cp(nx2, SBB - GO + bb, q).wait()
        if bb == 2:
          @pl.when(sb < SBN - 2)
          def _(nx2=nx2):
            stage_idx(sb + 2, nx2)
        for jg in range(KB // 16):
          av = adj_v[par, bb, pl.ds(jg * 16, 16)]
          for l in range(16):
            j = jg * 16 + l
            splat = _lane_splat(av, l)
            for blk in range(DH // 32):
              v = rin_v[gi, j, pl.ds(blk * 16, 16)]
              lo = lax.bitcast_convert_type(v << 16, jnp.float32)
              hi = lax.bitcast_convert_type(v & jnp.int32(-65536), jnp.float32)
              rout_v[q, j, pl.ds(blk * 32, 16)] = lo * splat
              rout_v[q, j, pl.ds(blk * 32 + 16, 16)] = hi * splat
        @pl.when(not_last)
        def _(nx1=nx1, bb=bb, gi=gi):
          gather_cp(nx1, bb, gi).start()
        scatter_cp(par, bb, q).start(add=True)
      @pl.when(sb < SBN - 2)
      def _(nx2=nx2):
        wait_idx(sb + 2, nx2)
        offset_src(nx2)
      return carry

    lax.fori_loop(0, SBN, sb_body, 0)

    last_par = (SBN - 1) % NIB
    for bb in range(SBB - GO, SBB):
      scatter_cp(last_par, bb, bb % GO).wait()

    plsc.subcore_barrier()
    pltpu.sync_copy(acc_sh.at[pl.ds(s * RPT, RPT)],
                    out_hbm.at[c, pl.ds(s * RPT, RPT)])

  return spmm_kernel(xflat, src_r, dst_r, adj_r, zeros_rows)


def _tc_combine_matmul(partials, weight, bias):
  bm = 2000
  grid = N_NODES_C // bm

  def body(p_ref, w_ref, b_ref, o_ref):
    agg = jnp.concatenate([p_ref[0], p_ref[1]], axis=-1)
    o_ref[...] = (
        jnp.dot(agg, w_ref[...], preferred_element_type=jnp.float32)
        + b_ref[...]
    )

  return pl.pallas_call(
      body,
      grid=(grid,),
      in_specs=[
          pl.BlockSpec((NC, bm, DH), lambda i: (0, i, 0)),
          pl.BlockSpec((D_C, D_C), lambda i: (0, 0)),
          pl.BlockSpec((1, D_C), lambda i: (0, 0)),
      ],
      out_specs=pl.BlockSpec((bm, D_C), lambda i: (i, 0)),
      out_shape=jax.ShapeDtypeStruct((N_NODES_C, D_C), jnp.float32),
  )(partials, weight, bias.reshape(1, D_C))


def kernel(x, edge_index, adj_values, weight, bias):
  ei = edge_index.astype(jnp.int32)
  src_r = ei[1].reshape(NS, SBN, SBE)
  dst_r = ei[0].reshape(NS, SBN, SBB, KB)
  adj_r = adj_values.reshape(NS, SBN, SBB, KB)
  xflat = jax.lax.bitcast_convert_type(
      x.astype(jnp.bfloat16).reshape(NC * N_NODES_C, DH // 2, 2),
      jnp.int32)
  zeros_rows = jnp.zeros((RPT, DH), jnp.float32)
  partials = _sc_spmm(xflat, src_r, dst_r, adj_r, zeros_rows)
  perm = jnp.array([h * 64 + b * 32 + 2 * k + e
                    for h in range(2) for b in range(2)
                    for e in range(2) for k in range(16)], jnp.int32)
  return _tc_combine_matmul(partials, weight[perm, :], bias)


# speedup vs baseline: 4.1923x; 4.1923x over previous
import functools

import jax
import jax.numpy as jnp
from jax import lax
from jax.experimental import pallas as pl
from jax.experimental.pallas import tpu as pltpu
from jax.experimental.pallas import tpu_sc as plsc

N_NODES_C = 10000
N_EDGES_C = 320000
D_C = 128
DH = D_C // 2

NC = 2
NS = 16
EPT = N_EDGES_C // NS
KB = 80
NBB = EPT // KB
N_PAD = 10240
RPT = N_PAD // NS
GI = 5
GO = 2
SBB = 10
SBN = NBB // SBB
SBE = SBB * KB


def _lane_splat(vec, lane):
  return lax.gather(
      vec, jnp.full((16, 1), lane, jnp.int32),
      dimension_numbers=lax.GatherDimensionNumbers(
          offset_dims=(), collapsed_slice_dims=(0,), start_index_map=(0,)),
      slice_sizes=(1,),
      mode=lax.GatherScatterMode.PROMISE_IN_BOUNDS)


def _sc_spmm(xflat, src_r, dst_r, adj_r, zeros_rows):
  mesh = plsc.VectorSubcoreMesh(core_axis_name="c", subcore_axis_name="s")

  @functools.partial(
      pl.kernel,
      mesh=mesh,
      compiler_params=pltpu.CompilerParams(use_tc_tiling_on_sc=False),
      out_type=jax.ShapeDtypeStruct((NC, N_PAD, DH), jnp.float32),
      scratch_types=[
          pltpu.VMEM((2, SBE), jnp.int32),
          pltpu.VMEM((2, SBB, KB), jnp.int32),
          pltpu.VMEM((2, SBB, KB), jnp.float32),
          pltpu.VMEM((GI, KB, DH), jnp.float32),
          pltpu.VMEM((GO, KB, DH), jnp.float32),
          pltpu.VMEM_SHARED((N_PAD, DH), jnp.float32),
      ] + [pltpu.SemaphoreType.DMA] * (GI + GO + 2),
  )
  def spmm_kernel(x_hbm, src_hbm, dst_hbm, adj_hbm, z_hbm, out_hbm,
                  src_v, dst_v, adj_v, rin_v, rout_v, acc_sh, *sems):
    c = lax.axis_index("c")
    s = lax.axis_index("s")
    gsem = sems[:GI]
    ssem = sems[GI:GI + GO]
    ksem = sems[GI + GO]
    zsem = sems[GI + GO + 1]

    offv = jnp.full((16,), c, jnp.int32)

    def stage_idx(sb, par):
      pltpu.make_async_copy(src_hbm.at[s, sb], src_v.at[par], ksem).start()
      pltpu.make_async_copy(dst_hbm.at[s, sb], dst_v.at[par], ksem).start()
      pltpu.make_async_copy(adj_hbm.at[s, sb], adj_v.at[par], ksem).start()

    def wait_idx(sb, par):
      pltpu.make_async_copy(src_hbm.at[s, sb], src_v.at[par], ksem).wait()
      pltpu.make_async_copy(dst_hbm.at[s, sb], dst_v.at[par], ksem).wait()
      pltpu.make_async_copy(adj_hbm.at[s, sb], adj_v.at[par], ksem).wait()

    def offset_src(par):
      def off_body(i, carry):
        sl = pl.ds(i * 16, 16)
        src_v[par, sl] = src_v[par, sl] * 2 + offv
        return carry
      lax.fori_loop(0, SBE // 16, off_body, 0)

    def gather_cp(par, bb, i):
      return pltpu.make_async_copy(
          x_hbm.at[src_v.at[par, pl.ds(bb * KB, KB)]], rin_v.at[i], gsem[i])

    def scatter_cp(par, bb, q):
      return pltpu.make_async_copy(rout_v.at[q], acc_sh.at[dst_v.at[par, bb]],
                                   ssem[q])

    zcp = pltpu.async_copy(z_hbm, acc_sh.at[pl.ds(s * RPT, RPT)], zsem)
    stage_idx(0, 0)
    wait_idx(0, 0)
    offset_src(0)
    for i in range(GI):
      gather_cp(0, i, i).start()
    zcp.wait()
    plsc.subcore_barrier()

    def sb_body(sb, carry):
      par = lax.rem(sb, 2)
      nxt = 1 - par
      not_last = sb < SBN - 1
      for bb in range(SBB):
        gi = bb % GI
        q = bb % GO
        gather_cp(par, bb, gi).wait()
        if bb >= GO:
          scatter_cp(par, bb - GO, q).wait()
        else:
          @pl.when(sb > 0)
          def _(par=par, bb=bb, q=q):
            scatter_cp(1 - par, SBB - GO + bb, q).wait()
        if bb == 2:
          @pl.when(not_last)
          def _(nxt=nxt):
            stage_idx(sb + 1, nxt)
        for jg in range(KB // 16):
          av = adj_v[par, bb, pl.ds(jg * 16, 16)]
          for l in range(16):
            j = jg * 16 + l
            splat = _lane_splat(av, l)
            for cc in range(DH // 16):
              sl = pl.ds(cc * 16, 16)
              rout_v[q, j, sl] = rin_v[gi, j, sl] * splat
        if bb == GI - 1:
          @pl.when(not_last)
          def _(nxt=nxt):
            wait_idx(sb + 1, nxt)
            offset_src(nxt)
        if bb < SBB - GI:
          gather_cp(par, bb + GI, gi).start()
        else:
          @pl.when(not_last)
          def _(nxt=nxt, bb=bb, gi=gi):
            gather_cp(nxt, bb - (SBB - GI), gi).start()
        scatter_cp(par, bb, q).start(add=True)
      return carry

    lax.fori_loop(0, SBN, sb_body, 0)

    last_par = (SBN - 1) % 2
    for bb in range(SBB - GO, SBB):
      scatter_cp(last_par, bb, bb % GO).wait()

    plsc.subcore_barrier()
    pltpu.sync_copy(acc_sh.at[pl.ds(s * RPT, RPT)],
                    out_hbm.at[c, pl.ds(s * RPT, RPT)])

  return spmm_kernel(xflat, src_r, dst_r, adj_r, zeros_rows)


def _tc_combine_matmul(partials, weight, bias):
  bm = 2000
  grid = N_NODES_C // bm

  def body(p_ref, w_ref, b_ref, o_ref):
    agg = jnp.concatenate([p_ref[0], p_ref[1]], axis=-1)
    o_ref[...] = (
        jnp.dot(agg, w_ref[...], preferred_element_type=jnp.float32)
        + b_ref[...]
    )

  return pl.pallas_call(
      body,
      grid=(grid,),
      in_specs=[
          pl.BlockSpec((NC, bm, DH), lambda i: (0, i, 0)),
          pl.BlockSpec((D_C, D_C), lambda i: (0, 0)),
          pl.BlockSpec((1, D_C), lambda i: (0, 0)),
      ],
      out_specs=pl.BlockSpec((bm, D_C), lambda i: (i, 0)),
      out_shape=jax.ShapeDtypeStruct((N_NODES_C, D_C), jnp.float32),
  )(partials, weight, bias.reshape(1, D_C))


def kernel(x, edge_index, adj_values, weight, bias):
  ei = edge_index.astype(jnp.int32)
  src_r = ei[1].reshape(NS, SBN, SBE)
  dst_r = ei[0].reshape(NS, SBN, SBB, KB)
  adj_r = adj_values.reshape(NS, SBN, SBB, KB)
  xflat = x.reshape(NC * N_NODES_C, DH)
  zeros_rows = jnp.zeros((RPT, DH), jnp.float32)
  partials = _sc_spmm(xflat, src_r, dst_r, adj_r, zeros_rows)
  return _tc_combine_matmul(partials, weight, bias)
